# baseline (speedup 1.0000x reference)
import functools

import jax
import jax.numpy as jnp
from jax import lax
from jax.experimental import pallas as pl
from jax.experimental.pallas import tpu as pltpu
from jax.experimental.pallas import tpu_sc as plsc

N_FIELDS = 26
K = 16
BATCH = 4096
FPAD = 32
DPAD = FPAD * K
NJ = DPAD // 128
HASH = 1000000

NC, NS = 2, 16
NW = NC * NS
BPW = BATCH // NW
NSTREAM = NJ * (BPW // K)
RPT = BPW * FPAD


NBLK = (HASH + 127) // 128
NFULL = NBLK - 1


def _sc_transpose(embT, tail2):
    mesh = plsc.VectorSubcoreMesh(core_axis_name="c", subcore_axis_name="s")

    @functools.partial(
        pl.kernel,
        mesh=mesh,
        out_type=jax.ShapeDtypeStruct((HASH // 8, 128), jnp.float32),
        scratch_types=[
            pltpu.VMEM((K, 128), jnp.float32),
            pltpu.VMEM((K, 128), jnp.float32),
            pltpu.SemaphoreType.DMA,
        ],
        compiler_params=pltpu.CompilerParams(needs_layout_passes=False),
    )
    def k(embT_hbm, tail_hbm, out_hbm, in_v, out_v, sem):
        wid = lax.axis_index("s") * NC + lax.axis_index("c")
        nper = NFULL // NW
        rem = NFULL - nper * NW

        iota = lax.iota(jnp.int32, 16)

        def do_block(c, _):
            pltpu.sync_copy(embT_hbm.at[:, pl.ds(c * 128, 128)], in_v)
            for s in range(16):
                for g in range(8):
                    col = jnp.full((16,), 8 * s + g, jnp.int32)
                    out_v[s, pl.ds(16 * g, 16)] = plsc.load_gather(
                        in_v, [iota, col])
            pltpu.sync_copy(out_v, out_hbm.at[pl.ds(c * 16, 16)])
            return 0

        lax.fori_loop(wid * nper, (wid + 1) * nper, do_block, 0)
        @pl.when(wid < rem)
        def _():
            lax.fori_loop(NW * nper + wid, NW * nper + wid + 1, do_block, 0)

        @pl.when(wid == NW - 1)
        def _():
            pltpu.sync_copy(tail_hbm, in_v.at[pl.ds(0, 8)])
            pltpu.sync_copy(in_v.at[pl.ds(0, 8)],
                            out_hbm.at[pl.ds((HASH // 8) - 8, 8)])

    return k(embT, tail2)


def _sc_gather(xq3, xp3, emb_v, w1):
    mesh = plsc.VectorSubcoreMesh(core_axis_name="c", subcore_axis_name="s")

    @functools.partial(
        pl.kernel,
        mesh=mesh,
        out_type=[jax.ShapeDtypeStruct((BATCH * 8, K), jnp.float32)
                  for _ in range(NJ)]
        + [jax.ShapeDtypeStruct((BATCH, FPAD), jnp.float32)],
        scratch_types=[
            pltpu.VMEM((NSTREAM, 128), jnp.int32),
            pltpu.VMEM((BPW, FPAD), jnp.int32),
            pltpu.VMEM((RPT, K), jnp.float32),
            pltpu.VMEM((BPW, FPAD), jnp.float32),
            pltpu.SemaphoreType.DMA,
            pltpu.SemaphoreType.DMA,
        ],
        compiler_params=pltpu.CompilerParams(use_tc_tiling_on_sc=False),
    )
    def k(xq_hbm, xp_hbm, emb_hbm, w1_hbm, d0_out, d1_out, d2_out, d3_out,
          w1g_out, xq_v, xp_v, stag_v, w1r_v, sem_e, sem_w):
        wid = lax.axis_index("s") * NC + lax.axis_index("c")
        pltpu.sync_copy(xq_hbm.at[wid], xq_v)
        pltpu.sync_copy(xp_hbm.at[wid], xp_v)

        def fire_e(s, _):
            pltpu.async_copy(emb_hbm.at[xq_v.at[s]],
                             stag_v.at[pl.ds(s * 128, 128)], sem_e)
            return 0

        lax.fori_loop(0, NSTREAM, fire_e, 0)

        def fire_w(b, _):
            pltpu.async_copy(w1_hbm.at[xp_v.at[b]], w1r_v.at[b], sem_w)
            return 0

        lax.fori_loop(0, BPW, fire_w, 0)
        pltpu.make_async_copy(
            d0_out.at[pl.ds(0, RPT)], stag_v, sem_e).wait()
        pltpu.make_async_copy(
            w1g_out.at[pl.ds(0, BPW)], w1r_v, sem_w).wait()
        qb = wid * (8 * BPW)
        pltpu.sync_copy(stag_v.at[pl.ds(0, 1024)], d0_out.at[pl.ds(qb, 1024)])
        pltpu.sync_copy(stag_v.at[pl.ds(1024, 1024)], d1_out.at[pl.ds(qb, 1024)])
        pltpu.sync_copy(stag_v.at[pl.ds(2048, 1024)], d2_out.at[pl.ds(qb, 1024)])
        pltpu.sync_copy(stag_v.at[pl.ds(3072, 1024)], d3_out.at[pl.ds(qb, 1024)])
        pltpu.sync_copy(w1r_v, w1g_out.at[pl.ds(wid * BPW, BPW)])

    return k(xq3, xp3, emb_v, w1)


BM = 512


def _tc_body(d0_ref, d1_ref, d2_ref, d3_ref, w1g_ref, w0_ref, W1_ref, b1_ref,
             W2_ref, b2_ref, Wout_ref, S_ref, out_ref):
    lane = lax.broadcasted_iota(jnp.int32, (1, 128), 1)
    d3m = jnp.where(lane < 32, d3_ref[...], 0.0)
    d = jnp.concatenate(
        [d0_ref[...], d1_ref[...], d2_ref[...], d3m], axis=1
    )
    sumV = jnp.dot(d, S_ref[...], preferred_element_type=jnp.float32)
    s2 = jnp.sum(sumV * sumV, axis=1, keepdims=True)
    sq = jnp.sum(d * d, axis=1, keepdims=True)
    fm2 = (s2 - sq) * 0.5
    lane32 = lax.broadcasted_iota(jnp.int32, (1, FPAD), 1)
    w1m = jnp.where(lane32 < N_FIELDS, w1g_ref[...], 0.0)
    fm1 = jnp.sum(w1m, axis=1, keepdims=True)
    h = jnp.maximum(
        jnp.dot(d, W1_ref[...], preferred_element_type=jnp.float32)
        + b1_ref[...], 0.0)
    h = jnp.maximum(
        jnp.dot(h, W2_ref[...], preferred_element_type=jnp.float32)
        + b2_ref[...], 0.0)
    logit = (jnp.dot(h, Wout_ref[...], preferred_element_type=jnp.float32)
             + w0_ref[...] + fm1 + fm2)
    out_ref[...] = 1.0 / (1.0 + jnp.exp(-logit))


def _tc_mlp(d0, d1, d2, d3, w1g, w0, W1p, b1, W2, b2, Wout, S):
    h1 = W1p.shape[1]
    h2 = W2.shape[1]
    dspec = pl.BlockSpec((BM, 128), lambda i: (i, 0))
    return pl.pallas_call(
        _tc_body,
        grid=(BATCH // BM,),
        in_specs=[
            dspec, dspec, dspec, dspec,
            pl.BlockSpec((BM, FPAD), lambda i: (i, 0)),
            pl.BlockSpec((1, 1), lambda i: (0, 0)),
            pl.BlockSpec((DPAD, h1), lambda i: (0, 0)),
            pl.BlockSpec((1, h1), lambda i: (0, 0)),
            pl.BlockSpec((h1, h2), lambda i: (0, 0)),
            pl.BlockSpec((1, h2), lambda i: (0, 0)),
            pl.BlockSpec((h2, 1), lambda i: (0, 0)),
            pl.BlockSpec((DPAD, K), lambda i: (0, 0)),
        ],
        out_specs=pl.BlockSpec((BM, 1), lambda i: (i, 0)),
        out_shape=jax.ShapeDtypeStruct((BATCH, 1), jnp.float32),
    )(d0, d1, d2, d3, w1g, w0, W1p, b1, W2, b2, Wout, S)


def kernel(x, emb_v, w0, w1, W_h1, b_h1, W_h2, b_h2, W_out):
    pad = (jax.lax.broadcasted_iota(jnp.int32, (FPAD - N_FIELDS, BATCH), 0)
           + jax.lax.broadcasted_iota(jnp.int32, (FPAD - N_FIELDS, BATCH), 1)
           * 13) % HASH
    xpad = jnp.concatenate([x, pad], axis=0)
    xq3 = (xpad.reshape(NJ, 8, NW, 8, K)
           .transpose(2, 0, 3, 4, 1)
           .reshape(NW, NSTREAM, 128))
    xp3 = xpad.T.reshape(NW, BPW, FPAD)
    embL = _sc_transpose(emb_v.T, emb_v[HASH - 64:].reshape(8, 128))
    embL2 = embL.reshape(HASH, K)
    d0, d1, d2, d3, w1g = _sc_gather(xq3, xp3, embL2, w1.reshape(-1))
    d0 = d0.reshape(BATCH, 128)
    d1 = d1.reshape(BATCH, 128)
    d2 = d2.reshape(BATCH, 128)
    d3 = d3.reshape(BATCH, 128)
    W1f = W_h1.reshape(N_FIELDS, K, -1)
    W1p = jnp.zeros((FPAD, K, W_h1.shape[1]), jnp.float32).at[:N_FIELDS].set(
        W1f).reshape(DPAD, -1)
    S = jnp.zeros((FPAD, K, K), jnp.float32).at[:N_FIELDS].set(
        jnp.broadcast_to(jnp.eye(K, dtype=jnp.float32), (N_FIELDS, K, K))
    ).reshape(DPAD, K)
    return _tc_mlp(d0, d1, d2, d3, w1g, jnp.reshape(w0, (1, 1)), W1p,
                   b_h1.reshape(1, -1), W_h2, b_h2.reshape(1, -1), W_out, S)


# speedup vs baseline: 1.4579x; 1.4579x over previous
import functools

import jax
import jax.numpy as jnp
from jax import lax
from jax.experimental import pallas as pl
from jax.experimental.pallas import tpu as pltpu
from jax.experimental.pallas import tpu_sc as plsc

N_FIELDS = 26
K = 16
BATCH = 4096
FPAD = 32
DPAD = FPAD * K
NJ = DPAD // 128
HASH = 1000000

NC, NS = 2, 16
NW = NC * NS
BPW = BATCH // NW
NSTREAM = NJ * (BPW // K)
RPT = BPW * FPAD


NBLK = (HASH + 127) // 128
NFULL = NBLK - 1
SUP = 4
NSUP = NFULL // SUP
SPW = NSUP // NW
SCOLS = SUP * 128


def _sc_transpose(embT, tail2):
    mesh = plsc.VectorSubcoreMesh(core_axis_name="c", subcore_axis_name="s")

    @functools.partial(
        pl.kernel,
        mesh=mesh,
        out_type=jax.ShapeDtypeStruct((HASH // 8, 128), jnp.float32),
        scratch_types=[
            pltpu.VMEM((K, SCOLS), jnp.float32),
            pltpu.VMEM((K, SCOLS), jnp.float32),
            pltpu.VMEM((SUP * 16, 128), jnp.float32),
            pltpu.VMEM((SUP * 16, 128), jnp.float32),
            pltpu.SemaphoreType.DMA,
            pltpu.SemaphoreType.DMA,
            pltpu.SemaphoreType.DMA,
            pltpu.SemaphoreType.DMA,
        ],
        compiler_params=pltpu.CompilerParams(needs_layout_passes=False),
    )
    def k(embT_hbm, tail_hbm, out_hbm, inA, inB, outA, outB,
          semrA, semrB, semwA, semwB):
        wid = lax.axis_index("s") * NC + lax.axis_index("c")
        iota = lax.iota(jnp.int32, 16)

        def gsup(s):
            return jnp.where(s >= SPW, NSUP - 1, wid * SPW + s)

        def start_read(s, buf, sem):
            pltpu.async_copy(
                embT_hbm.at[:, pl.ds(gsup(s) * SCOLS, SCOLS)], buf, sem)

        def transpose(in_v, out_v):
            def blk(b, _):
                base = b * 128
                for s16 in range(16):
                    for g in range(8):
                        col = jnp.full((16,), base + 8 * s16 + g, jnp.int32)
                        out_v[b * 16 + s16, pl.ds(16 * g, 16)] = (
                            plsc.load_gather(in_v, [iota, col]))
                return 0

            lax.fori_loop(0, SUP, blk, 0)

        def start_write(s, buf, sem):
            pltpu.async_copy(
                buf, out_hbm.at[pl.ds(gsup(s) * (SUP * 16), SUP * 16)], sem)

        def drain(buf, sem):
            pltpu.make_async_copy(
                buf, out_hbm.at[pl.ds(0, SUP * 16)], sem).wait()

        nsup_here = jnp.where(wid == 0, SPW + 1, SPW)
        start_read(0, inA, semrA)

        def pair(p, _):
            sA, sB, sA2 = 2 * p, 2 * p + 1, 2 * p + 2

            @pl.when(sB < nsup_here)
            def _():
                start_read(sB, inB, semrB)
            pltpu.make_async_copy(embT_hbm.at[:, pl.ds(0, SCOLS)],
                                  inA, semrA).wait()
            @pl.when(p > 0)
            def _():
                drain(outA, semwA)
            transpose(inA, outA)
            start_write(sA, outA, semwA)

            @pl.when(sA2 < nsup_here)
            def _():
                start_read(sA2, inA, semrA)

            @pl.when(sB < nsup_here)
            def _():
                pltpu.make_async_copy(embT_hbm.at[:, pl.ds(0, SCOLS)],
                                      inB, semrB).wait()
                @pl.when(p > 0)
                def _():
                    drain(outB, semwB)
                transpose(inB, outB)
                start_write(sB, outB, semwB)
            return 0

        lax.fori_loop(0, (SPW + 2) // 2, pair, 0)
        drain(outA, semwA)
        drain(outB, semwB)

        @pl.when(wid == NW - 1)
        def _():
            pltpu.sync_copy(tail_hbm, inA.at[pl.ds(0, 8), pl.ds(0, 128)])
            pltpu.sync_copy(inA.at[pl.ds(0, 8), pl.ds(0, 128)],
                            out_hbm.at[pl.ds((HASH // 8) - 8, 8)])

    return k(embT, tail2)


def _sc_gather(xq3, xp3, emb_v, w1):
    mesh = plsc.VectorSubcoreMesh(core_axis_name="c", subcore_axis_name="s")

    @functools.partial(
        pl.kernel,
        mesh=mesh,
        out_type=[jax.ShapeDtypeStruct((BATCH * 8, K), jnp.float32)
                  for _ in range(NJ)]
        + [jax.ShapeDtypeStruct((BATCH, FPAD), jnp.float32)],
        scratch_types=[
            pltpu.VMEM((NSTREAM, 128), jnp.int32),
            pltpu.VMEM((BPW, FPAD), jnp.int32),
            pltpu.VMEM((RPT, K), jnp.float32),
            pltpu.VMEM((BPW, FPAD), jnp.float32),
            pltpu.SemaphoreType.DMA,
            pltpu.SemaphoreType.DMA,
        ],
        compiler_params=pltpu.CompilerParams(use_tc_tiling_on_sc=False),
    )
    def k(xq_hbm, xp_hbm, emb_hbm, w1_hbm, d0_out, d1_out, d2_out, d3_out,
          w1g_out, xq_v, xp_v, stag_v, w1r_v, sem_e, sem_w):
        wid = lax.axis_index("s") * NC + lax.axis_index("c")
        pltpu.sync_copy(xq_hbm.at[wid], xq_v)
        pltpu.sync_copy(xp_hbm.at[wid], xp_v)

        def fire_e(s, _):
            pltpu.async_copy(emb_hbm.at[xq_v.at[s]],
                             stag_v.at[pl.ds(s * 128, 128)], sem_e)
            return 0

        lax.fori_loop(0, NSTREAM, fire_e, 0)

        def fire_w(b, _):
            pltpu.async_copy(w1_hbm.at[xp_v.at[b]], w1r_v.at[b], sem_w)
            return 0

        lax.fori_loop(0, BPW, fire_w, 0)
        pltpu.make_async_copy(
            d0_out.at[pl.ds(0, RPT)], stag_v, sem_e).wait()
        pltpu.make_async_copy(
            w1g_out.at[pl.ds(0, BPW)], w1r_v, sem_w).wait()
        qb = wid * (8 * BPW)
        pltpu.sync_copy(stag_v.at[pl.ds(0, 1024)], d0_out.at[pl.ds(qb, 1024)])
        pltpu.sync_copy(stag_v.at[pl.ds(1024, 1024)], d1_out.at[pl.ds(qb, 1024)])
        pltpu.sync_copy(stag_v.at[pl.ds(2048, 1024)], d2_out.at[pl.ds(qb, 1024)])
        pltpu.sync_copy(stag_v.at[pl.ds(3072, 1024)], d3_out.at[pl.ds(qb, 1024)])
        pltpu.sync_copy(w1r_v, w1g_out.at[pl.ds(wid * BPW, BPW)])

    return k(xq3, xp3, emb_v, w1)


BM = 512


def _tc_body(d0_ref, d1_ref, d2_ref, d3_ref, w1g_ref, w0_ref, W1_ref, b1_ref,
             W2_ref, b2_ref, Wout_ref, S_ref, out_ref):
    lane = lax.broadcasted_iota(jnp.int32, (1, 128), 1)
    d3m = jnp.where(lane < 32, d3_ref[...], 0.0)
    d = jnp.concatenate(
        [d0_ref[...], d1_ref[...], d2_ref[...], d3m], axis=1
    )
    sumV = jnp.dot(d, S_ref[...], preferred_element_type=jnp.float32)
    s2 = jnp.sum(sumV * sumV, axis=1, keepdims=True)
    sq = jnp.sum(d * d, axis=1, keepdims=True)
    fm2 = (s2 - sq) * 0.5
    lane32 = lax.broadcasted_iota(jnp.int32, (1, FPAD), 1)
    w1m = jnp.where(lane32 < N_FIELDS, w1g_ref[...], 0.0)
    fm1 = jnp.sum(w1m, axis=1, keepdims=True)
    h = jnp.maximum(
        jnp.dot(d, W1_ref[...], preferred_element_type=jnp.float32)
        + b1_ref[...], 0.0)
    h = jnp.maximum(
        jnp.dot(h, W2_ref[...], preferred_element_type=jnp.float32)
        + b2_ref[...], 0.0)
    logit = (jnp.dot(h, Wout_ref[...], preferred_element_type=jnp.float32)
             + w0_ref[...] + fm1 + fm2)
    out_ref[...] = 1.0 / (1.0 + jnp.exp(-logit))


def _tc_mlp(d0, d1, d2, d3, w1g, w0, W1p, b1, W2, b2, Wout, S):
    h1 = W1p.shape[1]
    h2 = W2.shape[1]
    dspec = pl.BlockSpec((BM, 128), lambda i: (i, 0))
    return pl.pallas_call(
        _tc_body,
        grid=(BATCH // BM,),
        in_specs=[
            dspec, dspec, dspec, dspec,
            pl.BlockSpec((BM, FPAD), lambda i: (i, 0)),
            pl.BlockSpec((1, 1), lambda i: (0, 0)),
            pl.BlockSpec((DPAD, h1), lambda i: (0, 0)),
            pl.BlockSpec((1, h1), lambda i: (0, 0)),
            pl.BlockSpec((h1, h2), lambda i: (0, 0)),
            pl.BlockSpec((1, h2), lambda i: (0, 0)),
            pl.BlockSpec((h2, 1), lambda i: (0, 0)),
            pl.BlockSpec((DPAD, K), lambda i: (0, 0)),
        ],
        out_specs=pl.BlockSpec((BM, 1), lambda i: (i, 0)),
        out_shape=jax.ShapeDtypeStruct((BATCH, 1), jnp.float32),
    )(d0, d1, d2, d3, w1g, w0, W1p, b1, W2, b2, Wout, S)


def kernel(x, emb_v, w0, w1, W_h1, b_h1, W_h2, b_h2, W_out):
    pad = (jax.lax.broadcasted_iota(jnp.int32, (FPAD - N_FIELDS, BATCH), 0)
           + jax.lax.broadcasted_iota(jnp.int32, (FPAD - N_FIELDS, BATCH), 1)
           * 13) % HASH
    xpad = jnp.concatenate([x, pad], axis=0)
    xq3 = (xpad.reshape(NJ, 8, NW, 8, K)
           .transpose(2, 0, 3, 4, 1)
           .reshape(NW, NSTREAM, 128))
    xp3 = xpad.T.reshape(NW, BPW, FPAD)
    embL = _sc_transpose(emb_v.T, emb_v[HASH - 64:].reshape(8, 128))
    embL2 = embL.reshape(HASH, K)
    d0, d1, d2, d3, w1g = _sc_gather(xq3, xp3, embL2, w1.reshape(-1))
    d0 = d0.reshape(BATCH, 128)
    d1 = d1.reshape(BATCH, 128)
    d2 = d2.reshape(BATCH, 128)
    d3 = d3.reshape(BATCH, 128)
    W1f = W_h1.reshape(N_FIELDS, K, -1)
    W1p = jnp.zeros((FPAD, K, W_h1.shape[1]), jnp.float32).at[:N_FIELDS].set(
        W1f).reshape(DPAD, -1)
    S = jnp.zeros((FPAD, K, K), jnp.float32).at[:N_FIELDS].set(
        jnp.broadcast_to(jnp.eye(K, dtype=jnp.float32), (N_FIELDS, K, K))
    ).reshape(DPAD, K)
    return _tc_mlp(d0, d1, d2, d3, w1g, jnp.reshape(w0, (1, 1)), W1p,
                   b_h1.reshape(1, -1), W_h2, b_h2.reshape(1, -1), W_out, S)


# speedup vs baseline: 1.5678x; 1.0753x over previous
import functools

import jax
import jax.numpy as jnp
from jax import lax
from jax.experimental import pallas as pl
from jax.experimental.pallas import tpu as pltpu
from jax.experimental.pallas import tpu_sc as plsc

N_FIELDS = 26
K = 16
BATCH = 4096
FPAD = 32
DPAD = FPAD * K
NJ = DPAD // 128
HASH = 1000000

NC, NS = 2, 16
NW = NC * NS
BPW = BATCH // NW
NSTREAM = NJ * (BPW // K)
RPT = BPW * FPAD


NBLK = (HASH + 127) // 128
NFULL = NBLK - 1
SUP = 4
NSUP = NFULL // SUP
SPW = NSUP // NW
SCOLS = SUP * 128


def _sc_transpose(embT, tail2):
    mesh = plsc.VectorSubcoreMesh(core_axis_name="c", subcore_axis_name="s")

    @functools.partial(
        pl.kernel,
        mesh=mesh,
        out_type=jax.ShapeDtypeStruct((HASH // 8, 128), jnp.float32),
        scratch_types=[
            pltpu.VMEM((K, SCOLS), jnp.float32),
            pltpu.VMEM((K, SCOLS), jnp.float32),
            pltpu.VMEM((SUP * 16, 128), jnp.float32),
            pltpu.VMEM((SUP * 16, 128), jnp.float32),
            pltpu.SemaphoreType.DMA,
            pltpu.SemaphoreType.DMA,
            pltpu.SemaphoreType.DMA,
            pltpu.SemaphoreType.DMA,
        ],
        compiler_params=pltpu.CompilerParams(needs_layout_passes=False),
    )
    def k(embT_hbm, tail_hbm, out_hbm, inA, inB, outA, outB,
          semrA, semrB, semwA, semwB):
        wid = lax.axis_index("s") * NC + lax.axis_index("c")
        iota = lax.iota(jnp.int32, 16)

        def gsup(s):
            return jnp.where(s >= SPW, NSUP - 1, wid * SPW + s)

        def start_read(s, buf, sem):
            pltpu.async_copy(
                embT_hbm.at[:, pl.ds(gsup(s) * SCOLS, SCOLS)], buf, sem)

        def transpose(in_v, out_v):
            def jj_body(jj, colv):
                for u in range(8):
                    out_v[jj, pl.ds(u * 16, 16)] = plsc.load_gather(
                        in_v, [iota, colv])
                    colv = colv + 1
                return colv

            lax.fori_loop(0, SUP * 16, jj_body,
                          jnp.zeros((16,), jnp.int32))

        def start_write(s, buf, sem):
            pltpu.async_copy(
                buf, out_hbm.at[pl.ds(gsup(s) * (SUP * 16), SUP * 16)], sem)

        def drain(buf, sem):
            pltpu.make_async_copy(
                buf, out_hbm.at[pl.ds(0, SUP * 16)], sem).wait()

        nsup_here = jnp.where(wid == 0, SPW + 1, SPW)
        start_read(0, inA, semrA)

        def pair(p, _):
            sA, sB, sA2 = 2 * p, 2 * p + 1, 2 * p + 2

            @pl.when(sB < nsup_here)
            def _():
                start_read(sB, inB, semrB)
            pltpu.make_async_copy(embT_hbm.at[:, pl.ds(0, SCOLS)],
                                  inA, semrA).wait()
            @pl.when(p > 0)
            def _():
                drain(outA, semwA)
            transpose(inA, outA)
            start_write(sA, outA, semwA)

            @pl.when(sA2 < nsup_here)
            def _():
                start_read(sA2, inA, semrA)

            @pl.when(sB < nsup_here)
            def _():
                pltpu.make_async_copy(embT_hbm.at[:, pl.ds(0, SCOLS)],
                                      inB, semrB).wait()
                @pl.when(p > 0)
                def _():
                    drain(outB, semwB)
                transpose(inB, outB)
                start_write(sB, outB, semwB)
            return 0

        lax.fori_loop(0, (SPW + 2) // 2, pair, 0)
        drain(outA, semwA)
        drain(outB, semwB)

        @pl.when(wid == NW - 1)
        def _():
            pltpu.sync_copy(tail_hbm, inA.at[pl.ds(0, 8), pl.ds(0, 128)])
            pltpu.sync_copy(inA.at[pl.ds(0, 8), pl.ds(0, 128)],
                            out_hbm.at[pl.ds((HASH // 8) - 8, 8)])

    return k(embT, tail2)


def _sc_gather(xq3, xp3, emb_v, w1):
    mesh = plsc.VectorSubcoreMesh(core_axis_name="c", subcore_axis_name="s")

    @functools.partial(
        pl.kernel,
        mesh=mesh,
        out_type=[jax.ShapeDtypeStruct((BATCH * 8, K), jnp.float32)
                  for _ in range(NJ)]
        + [jax.ShapeDtypeStruct((BATCH, FPAD), jnp.float32)],
        scratch_types=[
            pltpu.VMEM((NSTREAM, 128), jnp.int32),
            pltpu.VMEM((BPW, FPAD), jnp.int32),
            pltpu.VMEM((RPT, K), jnp.float32),
            pltpu.VMEM((BPW, FPAD), jnp.float32),
            pltpu.SemaphoreType.DMA,
            pltpu.SemaphoreType.DMA,
        ],
        compiler_params=pltpu.CompilerParams(use_tc_tiling_on_sc=False),
    )
    def k(xq_hbm, xp_hbm, emb_hbm, w1_hbm, d0_out, d1_out, d2_out, d3_out,
          w1g_out, xq_v, xp_v, stag_v, w1r_v, sem_e, sem_w):
        wid = lax.axis_index("s") * NC + lax.axis_index("c")
        pltpu.sync_copy(xq_hbm.at[wid], xq_v)
        pltpu.sync_copy(xp_hbm.at[wid], xp_v)

        def fire_e(s, _):
            pltpu.async_copy(emb_hbm.at[xq_v.at[s]],
                             stag_v.at[pl.ds(s * 128, 128)], sem_e)
            return 0

        lax.fori_loop(0, NSTREAM, fire_e, 0)

        def fire_w(b, _):
            pltpu.async_copy(w1_hbm.at[xp_v.at[b]], w1r_v.at[b], sem_w)
            return 0

        lax.fori_loop(0, BPW, fire_w, 0)
        pltpu.make_async_copy(
            d0_out.at[pl.ds(0, RPT)], stag_v, sem_e).wait()
        pltpu.make_async_copy(
            w1g_out.at[pl.ds(0, BPW)], w1r_v, sem_w).wait()
        qb = wid * (8 * BPW)
        pltpu.sync_copy(stag_v.at[pl.ds(0, 1024)], d0_out.at[pl.ds(qb, 1024)])
        pltpu.sync_copy(stag_v.at[pl.ds(1024, 1024)], d1_out.at[pl.ds(qb, 1024)])
        pltpu.sync_copy(stag_v.at[pl.ds(2048, 1024)], d2_out.at[pl.ds(qb, 1024)])
        pltpu.sync_copy(stag_v.at[pl.ds(3072, 1024)], d3_out.at[pl.ds(qb, 1024)])
        pltpu.sync_copy(w1r_v, w1g_out.at[pl.ds(wid * BPW, BPW)])

    return k(xq3, xp3, emb_v, w1)


BM = 512


def _tc_body(d0_ref, d1_ref, d2_ref, d3_ref, w1g_ref, w0_ref, W1_ref, b1_ref,
             W2_ref, b2_ref, Wout_ref, S_ref, out_ref):
    lane = lax.broadcasted_iota(jnp.int32, (1, 128), 1)
    d3m = jnp.where(lane < 32, d3_ref[...], 0.0)
    d = jnp.concatenate(
        [d0_ref[...], d1_ref[...], d2_ref[...], d3m], axis=1
    )
    sumV = jnp.dot(d, S_ref[...], preferred_element_type=jnp.float32)
    s2 = jnp.sum(sumV * sumV, axis=1, keepdims=True)
    sq = jnp.sum(d * d, axis=1, keepdims=True)
    fm2 = (s2 - sq) * 0.5
    lane32 = lax.broadcasted_iota(jnp.int32, (1, FPAD), 1)
    w1m = jnp.where(lane32 < N_FIELDS, w1g_ref[...], 0.0)
    fm1 = jnp.sum(w1m, axis=1, keepdims=True)
    h = jnp.maximum(
        jnp.dot(d, W1_ref[...], preferred_element_type=jnp.float32)
        + b1_ref[...], 0.0)
    h = jnp.maximum(
        jnp.dot(h, W2_ref[...], preferred_element_type=jnp.float32)
        + b2_ref[...], 0.0)
    logit = (jnp.dot(h, Wout_ref[...], preferred_element_type=jnp.float32)
             + w0_ref[...] + fm1 + fm2)
    out_ref[...] = 1.0 / (1.0 + jnp.exp(-logit))


def _tc_mlp(d0, d1, d2, d3, w1g, w0, W1p, b1, W2, b2, Wout, S):
    h1 = W1p.shape[1]
    h2 = W2.shape[1]
    dspec = pl.BlockSpec((BM, 128), lambda i: (i, 0))
    return pl.pallas_call(
        _tc_body,
        grid=(BATCH // BM,),
        in_specs=[
            dspec, dspec, dspec, dspec,
            pl.BlockSpec((BM, FPAD), lambda i: (i, 0)),
            pl.BlockSpec((1, 1), lambda i: (0, 0)),
            pl.BlockSpec((DPAD, h1), lambda i: (0, 0)),
            pl.BlockSpec((1, h1), lambda i: (0, 0)),
            pl.BlockSpec((h1, h2), lambda i: (0, 0)),
            pl.BlockSpec((1, h2), lambda i: (0, 0)),
            pl.BlockSpec((h2, 1), lambda i: (0, 0)),
            pl.BlockSpec((DPAD, K), lambda i: (0, 0)),
        ],
        out_specs=pl.BlockSpec((BM, 1), lambda i: (i, 0)),
        out_shape=jax.ShapeDtypeStruct((BATCH, 1), jnp.float32),
    )(d0, d1, d2, d3, w1g, w0, W1p, b1, W2, b2, Wout, S)


def kernel(x, emb_v, w0, w1, W_h1, b_h1, W_h2, b_h2, W_out):
    pad = (jax.lax.broadcasted_iota(jnp.int32, (FPAD - N_FIELDS, BATCH), 0)
           + jax.lax.broadcasted_iota(jnp.int32, (FPAD - N_FIELDS, BATCH), 1)
           * 13) % HASH
    xpad = jnp.concatenate([x, pad], axis=0)
    xq3 = (xpad.reshape(NJ, 8, NW, 8, K)
           .transpose(2, 0, 3, 4, 1)
           .reshape(NW, NSTREAM, 128))
    xp3 = xpad.T.reshape(NW, BPW, FPAD)
    embL = _sc_transpose(emb_v.T, emb_v[HASH - 64:].reshape(8, 128))
    embL2 = embL.reshape(HASH, K)
    d0, d1, d2, d3, w1g = _sc_gather(xq3, xp3, embL2, w1.reshape(-1))
    d0 = d0.reshape(BATCH, 128)
    d1 = d1.reshape(BATCH, 128)
    d2 = d2.reshape(BATCH, 128)
    d3 = d3.reshape(BATCH, 128)
    W1f = W_h1.reshape(N_FIELDS, K, -1)
    W1p = jnp.zeros((FPAD, K, W_h1.shape[1]), jnp.float32).at[:N_FIELDS].set(
        W1f).reshape(DPAD, -1)
    S = jnp.zeros((FPAD, K, K), jnp.float32).at[:N_FIELDS].set(
        jnp.broadcast_to(jnp.eye(K, dtype=jnp.float32), (N_FIELDS, K, K))
    ).reshape(DPAD, K)
    return _tc_mlp(d0, d1, d2, d3, w1g, jnp.reshape(w0, (1, 1)), W1p,
                   b_h1.reshape(1, -1), W_h2, b_h2.reshape(1, -1), W_out, S)
